# baseline (speedup 1.0000x reference)
import functools

import jax
import jax.numpy as jnp
from jax import lax
from jax.experimental import pallas as pl
from jax.experimental.compute_on import compute_on
from jax.experimental.pallas import tpu as pltpu
from jax.experimental.pallas import tpu_sc as plsc

TOP_K = 8
N_GROUPS = 64
NC, NS, LANES = 2, 16, 16
NW = NC * NS


def _scores_body(x_ref, w_ref, b_ref, s_ref):
    x_blk = x_ref[...]
    w = w_ref[...]
    logits = lax.dot_general(x_blk, w, (((1,), (1,)), ((), ())))
    logits = logits + b_ref[...]
    e = jnp.exp(logits - jnp.max(logits, axis=1, keepdims=True))
    s_ref[...] = e / jnp.sum(e, axis=1, keepdims=True)


def _tc_scores(xf, weight, b2, bt):
    tokens = xf.shape[0]
    h = xf.shape[1]
    return pl.pallas_call(
        _scores_body,
        grid=(tokens // bt,),
        in_specs=[
            pl.BlockSpec((bt, h), lambda i: (i, 0)),
            pl.BlockSpec((N_GROUPS, h), lambda i: (0, 0)),
            pl.BlockSpec((1, N_GROUPS), lambda i: (0, 0)),
        ],
        out_specs=pl.BlockSpec((bt, N_GROUPS), lambda i: (i, 0)),
        out_shape=jax.ShapeDtypeStruct((tokens, N_GROUPS), jnp.float32),
    )(xf, weight, b2)


def _make_sc_topk(chunk_tokens):
    tpw = chunk_tokens // NW
    groups = tpw // LANES

    @functools.partial(
        pl.kernel,
        out_type=[
            jax.ShapeDtypeStruct((chunk_tokens * TOP_K,), jnp.int32),
            jax.ShapeDtypeStruct((chunk_tokens * TOP_K,), jnp.float32),
        ],
        mesh=plsc.VectorSubcoreMesh(
            core_axis_name="c", subcore_axis_name="s",
            num_cores=NC, num_subcores=NS,
        ),
        compiler_params=pltpu.CompilerParams(needs_layout_passes=False),
        scratch_types=[
            pltpu.VMEM((tpw * N_GROUPS,), jnp.float32),
            pltpu.VMEM((tpw * TOP_K,), jnp.int32),
            pltpu.VMEM((tpw * TOP_K,), jnp.float32),
        ],
    )
    def sc_topk(scores_hbm, idx_hbm, wgt_hbm, sv, iv, wv):
        wid = lax.axis_index("c") * NS + lax.axis_index("s")
        base = wid * tpw
        pltpu.sync_copy(scores_hbm.at[pl.ds(base * N_GROUPS, tpw * N_GROUPS)], sv)

        ilv = 2

        def group_body(it, _):
            t_iota = lax.iota(jnp.int32, LANES)
            toks, fis = [], []
            for p in range(ilv):
                tok = (it * ilv + p) * LANES + t_iota
                toks.append(tok)
                fis.append(tok * N_GROUPS)
            sval = [[jnp.full((LANES,), -1, jnp.int32) for _ in range(TOP_K)]
                    for _ in range(ilv)]
            sidx = [[jnp.zeros((LANES,), jnp.int32) for _ in range(TOP_K)]
                    for _ in range(ilv)]
            for e in range(N_GROUPS - 1, -1, -1):
                for p in range(ilv):
                    cv = plsc.bitcast(plsc.load_gather(sv, [fis[p] + e]),
                                      jnp.int32)
                    ci = jnp.full((LANES,), e, jnp.int32)
                    for j in range(TOP_K):
                        c = cv >= sval[p][j]
                        nv = jnp.maximum(cv, sval[p][j])
                        if j < TOP_K - 1:
                            cv = jnp.minimum(cv, sval[p][j])
                            nci = jnp.where(c, sidx[p][j], ci)
                        ni = jnp.where(c, ci, sidx[p][j])
                        sval[p][j] = nv
                        sidx[p][j] = ni
                        if j < TOP_K - 1:
                            ci = nci
            for p in range(ilv):
                fval = [plsc.bitcast(v, jnp.float32) for v in sval[p]]
                denom = fval[0]
                for j in range(1, TOP_K):
                    denom = denom + fval[j]
                denom = denom + 1e-20
                pos = toks[p] * TOP_K
                for j in range(TOP_K):
                    plsc.store_scatter(iv, [pos + j], sidx[p][j])
                    plsc.store_scatter(wv, [pos + j], fval[j] / denom)
            return _

        lax.fori_loop(0, groups // ilv, group_body, None)
        pltpu.sync_copy(iv, idx_hbm.at[pl.ds(base * TOP_K, tpw * TOP_K)])
        pltpu.sync_copy(wv, wgt_hbm.at[pl.ds(base * TOP_K, tpw * TOP_K)])

    return sc_topk


def kernel(x, weight, bias):
    bsz, seq_len, h = x.shape
    tokens = bsz * seq_len
    xf = x.reshape(tokens, h)
    b2 = bias.reshape(1, N_GROUPS)

    n_chunks = 4
    ct = tokens // n_chunks
    sc_topk = _make_sc_topk(ct)

    score_parts = []
    for c in range(n_chunks):
        xc = lax.slice_in_dim(xf, c * ct, (c + 1) * ct, axis=0)
        score_parts.append(_tc_scores(xc, weight, b2, bt=1024))
    idx_parts, wgt_parts = [], []
    for c in range(n_chunks):
        with compute_on("tpu_sparsecore"):
            idx_c, wgt_c = sc_topk(score_parts[c].reshape(ct * N_GROUPS))
        idx_parts.append(idx_c.reshape(ct, TOP_K))
        wgt_parts.append(wgt_c.reshape(ct, TOP_K))
    idx_out = jnp.concatenate(idx_parts, axis=0)
    wgt_out = jnp.concatenate(wgt_parts, axis=0)
    aux_loss = jnp.asarray(0.0, dtype=jnp.float32)
    return (idx_out, wgt_out, aux_loss)


# speedup vs baseline: 2.0255x; 2.0255x over previous
import jax
import jax.numpy as jnp
from jax import lax
from jax.experimental import pallas as pl

TOP_K = 8
N_GROUPS = 64
NEG_INF = float("-inf")


def _gate_body(x_ref, w_ref, b_ref, idx_ref, wgt_ref):
    x_blk = x_ref[...]
    w = w_ref[...]
    logits = lax.dot_general(x_blk, w, (((1,), (1,)), ((), ())))
    logits = logits + b_ref[...]

    bt = logits.shape[0]
    e = jnp.exp(logits - jnp.max(logits, axis=1, keepdims=True))
    scores = e / jnp.sum(e, axis=1, keepdims=True)

    cols = lax.broadcasted_iota(jnp.int32, (bt, N_GROUPS), 1)
    vals = []
    s = scores
    for k in range(TOP_K):
        m = jnp.max(s, axis=1, keepdims=True)
        hit = s == m
        idx = jnp.min(jnp.where(hit, cols, N_GROUPS), axis=1, keepdims=True)
        idx_ref[:, k : k + 1] = idx
        vals.append(m)
        s = jnp.where(cols == idx, NEG_INF, s)
    v = jnp.concatenate(vals, axis=1)
    wgt_ref[...] = v / (jnp.sum(v, axis=1, keepdims=True) + 1e-20)


def kernel(x, weight, bias):
    bsz, seq_len, h = x.shape
    tokens = bsz * seq_len
    xf = x.reshape(tokens, h)
    b2 = bias.reshape(1, N_GROUPS)

    bt = 2048
    grid = (tokens // bt,)
    idx_out, wgt_out = pl.pallas_call(
        _gate_body,
        grid=grid,
        in_specs=[
            pl.BlockSpec((bt, h), lambda i: (i, 0)),
            pl.BlockSpec((N_GROUPS, h), lambda i: (0, 0)),
            pl.BlockSpec((1, N_GROUPS), lambda i: (0, 0)),
        ],
        out_specs=[
            pl.BlockSpec((bt, TOP_K), lambda i: (i, 0)),
            pl.BlockSpec((bt, TOP_K), lambda i: (i, 0)),
        ],
        out_shape=[
            jax.ShapeDtypeStruct((tokens, TOP_K), jnp.int32),
            jax.ShapeDtypeStruct((tokens, TOP_K), jnp.float32),
        ],
    )(xf, weight, b2)
    aux_loss = jnp.asarray(0.0, dtype=jnp.float32)
    return (idx_out, wgt_out, aux_loss)
